# baseline (speedup 1.0000x reference)
import functools

import jax
import jax.numpy as jnp
from jax import lax
from jax.experimental import pallas as pl
from jax.experimental.pallas import tpu as pltpu
from jax.experimental.pallas import tpu_sc as plsc

N_NODES = 10000
D = 128
D_H = 256
NC, NS = 2, 16
NW = NC * NS
GROUP = 128
GC = 80
E_PAD = NW * GC * GROUP
N_PAD = 10240
RPT = N_PAD // NS
CW = 128


@functools.lru_cache(maxsize=None)
def _make_seg_sum():
    mesh = plsc.VectorSubcoreMesh(core_axis_name="c", subcore_axis_name="s")

    def body(src_h, dst_h, tab_h, out_h, src_v, dst_v, rows_v, acc_sh):
        c = lax.axis_index("c")
        s = lax.axis_index("s")
        wid = c * NS + s

        zero16 = jnp.zeros((16,), jnp.float32)

        @pl.loop(0, GROUP)
        def _(i):
            @pl.loop(0, D // 16)
            def _(j):
                rows_v[i, pl.ds(j * 16, 16)] = zero16

        @pl.loop(0, RPT // GROUP)
        def _(j):
            base = s * RPT + j * GROUP
            pltpu.sync_copy(rows_v, acc_sh.at[pl.ds(base, GROUP)])

        plsc.subcore_barrier()

        pltpu.sync_copy(src_h.at[wid], src_v)
        pltpu.sync_copy(dst_h.at[wid], dst_v)

        @pl.loop(0, GC)
        def _(g):
            pltpu.sync_copy(tab_h.at[src_v.at[g]], rows_v)
            pltpu.sync_copy(rows_v, acc_sh.at[dst_v.at[g]], add=True)

        plsc.subcore_barrier()

        @pl.loop(0, RPT // GROUP)
        def _(j):
            base = s * RPT + j * GROUP
            pltpu.sync_copy(acc_sh.at[pl.ds(base, GROUP)],
                            out_h.at[c].at[pl.ds(base, GROUP)])

    return pl.kernel(
        body,
        out_type=jax.ShapeDtypeStruct((NC, N_PAD, D), jnp.float32),
        mesh=mesh,
        scratch_types=[
            pltpu.VMEM((GC, GROUP), jnp.int32),
            pltpu.VMEM((GC, GROUP), jnp.int32),
            pltpu.VMEM((GROUP, D), jnp.float32),
            pltpu.VMEM_SHARED((N_PAD, D), jnp.float32),
        ])


@functools.lru_cache(maxsize=None)
def _make_counts():
    mesh = plsc.VectorSubcoreMesh(core_axis_name="c", subcore_axis_name="s")

    def body(dst_h, cnt_h, dst_v, ones_v, cnt_sh):
        c = lax.axis_index("c")
        s = lax.axis_index("s")
        wid = c * NS + s

        @pl.loop(0, GROUP)
        def _(i):
            @pl.loop(0, CW // 16)
            def _(j):
                ones_v[i, pl.ds(j * 16, 16)] = jnp.zeros((16,), jnp.float32)

        @pl.loop(0, RPT // GROUP)
        def _(j):
            base = s * RPT + j * GROUP
            pltpu.sync_copy(ones_v, cnt_sh.at[pl.ds(base, GROUP)])

        @pl.loop(0, GROUP)
        def _(i):
            @pl.loop(0, CW // 16)
            def _(j):
                ones_v[i, pl.ds(j * 16, 16)] = jnp.ones((16,), jnp.float32)

        plsc.subcore_barrier()

        pltpu.sync_copy(dst_h.at[wid], dst_v)

        @pl.loop(0, GC)
        def _(g):
            pltpu.sync_copy(ones_v, cnt_sh.at[dst_v.at[g]], add=True)

        plsc.subcore_barrier()

        @pl.loop(0, RPT // GROUP)
        def _(j):
            base = s * RPT + j * GROUP
            pltpu.sync_copy(cnt_sh.at[pl.ds(base, GROUP)],
                            cnt_h.at[c].at[pl.ds(base, GROUP)])

    return pl.kernel(
        body,
        out_type=jax.ShapeDtypeStruct((NC, N_PAD, CW), jnp.float32),
        mesh=mesh,
        scratch_types=[
            pltpu.VMEM((GC, GROUP), jnp.int32),
            pltpu.VMEM((GROUP, CW), jnp.float32),
            pltpu.VMEM_SHARED((N_PAD, CW), jnp.float32),
        ])


BLK = 1024
NBLK = N_PAD // BLK

_DOT = functools.partial(jnp.dot, preferred_element_type=jnp.float32,
                         precision=jax.lax.Precision.HIGHEST)


def _tc1_body(s0, s1, c0, c1, x, w1l, b1, w1r, w2l, h_ref, y_ref):
    cnt = jnp.maximum(c0[...][:, 0:1] + c1[...][:, 0:1], 1.0)
    mean = (s0[...] + s1[...]) / cnt
    acc = _DOT(mean, w1l[...]) + _DOT(x[...], w1r[...]) + b1[...]
    h = jnp.maximum(acc, 0.0)
    h_ref[...] = h
    y_ref[...] = _DOT(h, w2l[...])


def _tc2_body(t0, t1, c0, c1, h, w2r, b2, o_ref):
    cnt = jnp.maximum(c0[...][:, 0:1] + c1[...][:, 0:1], 1.0)
    mean = (t0[...] + t1[...]) / cnt
    o_ref[...] = mean + b2[...] + _DOT(h[...], w2r[...])


def _row_spec(w):
    return pl.BlockSpec((BLK, w), lambda i: (i, 0))


def _full_spec(shape):
    return pl.BlockSpec(shape, lambda i: (0,) * len(shape))


_tc1 = pl.pallas_call(
    _tc1_body,
    grid=(NBLK,),
    in_specs=[
        _row_spec(D), _row_spec(D), _row_spec(CW), _row_spec(CW), _row_spec(D),
        _full_spec((D, D_H)), _full_spec((1, D_H)), _full_spec((D, D_H)),
        _full_spec((D_H, D)),
    ],
    out_specs=[_row_spec(D_H), _row_spec(D)],
    out_shape=[
        jax.ShapeDtypeStruct((N_PAD, D_H), jnp.float32),
        jax.ShapeDtypeStruct((N_PAD, D), jnp.float32),
    ],
)

_tc2 = pl.pallas_call(
    _tc2_body,
    grid=(NBLK,),
    in_specs=[
        _row_spec(D), _row_spec(D), _row_spec(CW), _row_spec(CW),
        _row_spec(D_H), _full_spec((D_H, D)), _full_spec((1, D)),
    ],
    out_specs=_row_spec(D),
    out_shape=jax.ShapeDtypeStruct((N_PAD, D), jnp.float32),
)


def kernel(x, edge_index, W1_l, b1_l, W1_r, W2_l, b2_l, W2_r):
    src = edge_index[0]
    dst = edge_index[1]
    pad = E_PAD - src.shape[0]
    src_p = jnp.concatenate([src, jnp.zeros((pad,), jnp.int32)])
    dst_p = jnp.concatenate([dst, jnp.full((pad,), N_NODES, jnp.int32)])
    src3 = src_p.reshape(NW, GC, GROUP)
    dst3 = dst_p.reshape(NW, GC, GROUP)
    x_pad = jnp.zeros((N_PAD, D), jnp.float32).at[:N_NODES].set(x)

    cnts = _make_counts()(dst3)
    s1 = _make_seg_sum()(src3, dst3, x)
    h_pad, y_pad = _tc1(s1[0], s1[1], cnts[0], cnts[1], x_pad,
                        W1_l.T, b1_l.reshape(1, -1), W1_r.T, W2_l.T)
    s2 = _make_seg_sum()(src3, dst3, y_pad)
    out_pad = _tc2(s2[0], s2[1], cnts[0], cnts[1], h_pad,
                   W2_r.T, b2_l.reshape(1, -1))
    return out_pad[:N_NODES]


# speedup vs baseline: 2.5345x; 2.5345x over previous
import functools

import jax
import jax.numpy as jnp
from jax import lax
from jax.experimental import pallas as pl
from jax.experimental.pallas import tpu as pltpu
from jax.experimental.pallas import tpu_sc as plsc

N_NODES = 10000
D = 128
D_H = 256
NC, NS = 2, 16
NW = NC * NS
GROUP = 128
GC = 80
E_PAD = NW * GC * GROUP
N_PAD = 10240
RPT = N_PAD // NS
CW = 128


@functools.lru_cache(maxsize=None)
def _make_seg_sum():
    mesh = plsc.VectorSubcoreMesh(core_axis_name="c", subcore_axis_name="s")

    def body(src_h, dst_h, tab_h, out_h, src_v, dst_v, rows_v, acc_sh):
        c = lax.axis_index("c")
        s = lax.axis_index("s")
        wid = c * NS + s

        zero16 = jnp.zeros((16,), jnp.float32)

        @pl.loop(0, GROUP)
        def _(i):
            @pl.loop(0, D // 16)
            def _(j):
                rows_v[i, pl.ds(j * 16, 16)] = zero16

        @pl.loop(0, RPT // GROUP)
        def _(j):
            base = s * RPT + j * GROUP
            pltpu.sync_copy(rows_v, acc_sh.at[pl.ds(base, GROUP)])

        plsc.subcore_barrier()

        pltpu.sync_copy(src_h.at[wid], src_v)
        pltpu.sync_copy(dst_h.at[wid], dst_v)

        @pl.loop(0, GC)
        def _(g):
            pltpu.sync_copy(tab_h.at[src_v.at[g]], rows_v)
            pltpu.sync_copy(rows_v, acc_sh.at[dst_v.at[g]], add=True)

        plsc.subcore_barrier()

        @pl.loop(0, RPT // GROUP)
        def _(j):
            base = s * RPT + j * GROUP
            pltpu.sync_copy(acc_sh.at[pl.ds(base, GROUP)],
                            out_h.at[c].at[pl.ds(base, GROUP)])

    return pl.kernel(
        body,
        out_type=jax.ShapeDtypeStruct((NC, N_PAD, D), jnp.float32),
        mesh=mesh,
        scratch_types=[
            pltpu.VMEM((GC, GROUP), jnp.int32),
            pltpu.VMEM((GC, GROUP), jnp.int32),
            pltpu.VMEM((GROUP, D), jnp.float32),
            pltpu.VMEM_SHARED((N_PAD, D), jnp.float32),
        ])


@functools.lru_cache(maxsize=None)
def _make_counts():
    mesh = plsc.VectorSubcoreMesh(core_axis_name="c", subcore_axis_name="s")

    def body(dst_h, cnt_h, dst_v, ones_v, cnt_sh):
        c = lax.axis_index("c")
        s = lax.axis_index("s")
        wid = c * NS + s

        @pl.loop(0, GROUP)
        def _(i):
            @pl.loop(0, CW // 16)
            def _(j):
                ones_v[i, pl.ds(j * 16, 16)] = jnp.zeros((16,), jnp.float32)

        @pl.loop(0, RPT // GROUP)
        def _(j):
            base = s * RPT + j * GROUP
            pltpu.sync_copy(ones_v, cnt_sh.at[pl.ds(base, GROUP)])

        @pl.loop(0, GROUP)
        def _(i):
            @pl.loop(0, CW // 16)
            def _(j):
                ones_v[i, pl.ds(j * 16, 16)] = jnp.ones((16,), jnp.float32)

        plsc.subcore_barrier()

        pltpu.sync_copy(dst_h.at[wid], dst_v)

        @pl.loop(0, GC)
        def _(g):
            pltpu.sync_copy(ones_v, cnt_sh.at[dst_v.at[g]], add=True)

        plsc.subcore_barrier()

        @pl.loop(0, RPT // GROUP)
        def _(j):
            base = s * RPT + j * GROUP
            pltpu.sync_copy(cnt_sh.at[pl.ds(base, GROUP)],
                            cnt_h.at[c].at[pl.ds(base, GROUP)])

    return pl.kernel(
        body,
        out_type=jax.ShapeDtypeStruct((NC, N_PAD, CW), jnp.float32),
        mesh=mesh,
        scratch_types=[
            pltpu.VMEM((GC, GROUP), jnp.int32),
            pltpu.VMEM((GROUP, CW), jnp.float32),
            pltpu.VMEM_SHARED((N_PAD, CW), jnp.float32),
        ])


BLK = 1024
NBLK = N_PAD // BLK

_DOT = functools.partial(jnp.dot, preferred_element_type=jnp.float32,
                         precision=jax.lax.Precision.HIGHEST)


def _tc1_body(s0, s1, c0, c1, x, w1l, b1, w1r, w2l, h_ref, y_ref):
    cnt = jnp.maximum(c0[...][:, 0:1] + c1[...][:, 0:1], 1.0)
    mean = (s0[...] + s1[...]) / cnt
    acc = _DOT(mean, w1l[...]) + _DOT(x[...], w1r[...]) + b1[...]
    h = jnp.maximum(acc, 0.0)
    h_ref[...] = h
    y_ref[...] = _DOT(h, w2l[...])


def _tc2_body(t0, t1, c0, c1, h, w2r, b2, o_ref):
    cnt = jnp.maximum(c0[...][:, 0:1] + c1[...][:, 0:1], 1.0)
    mean = (t0[...] + t1[...]) / cnt
    o_ref[...] = mean + b2[...] + _DOT(h[...], w2r[...])


def _row_spec(w):
    return pl.BlockSpec((BLK, w), lambda i: (i, 0))


def _full_spec(shape):
    return pl.BlockSpec(shape, lambda i: (0,) * len(shape))


_tc1 = pl.pallas_call(
    _tc1_body,
    grid=(NBLK,),
    in_specs=[
        _row_spec(D), _row_spec(D), _row_spec(CW), _row_spec(CW), _row_spec(D),
        _full_spec((D, D_H)), _full_spec((1, D_H)), _full_spec((D, D_H)),
        _full_spec((D_H, D)),
    ],
    out_specs=[_row_spec(D_H), _row_spec(D)],
    out_shape=[
        jax.ShapeDtypeStruct((N_PAD, D_H), jnp.float32),
        jax.ShapeDtypeStruct((N_PAD, D), jnp.float32),
    ],
)

_tc2 = pl.pallas_call(
    _tc2_body,
    grid=(NBLK,),
    in_specs=[
        _row_spec(D), _row_spec(D), _row_spec(CW), _row_spec(CW),
        _row_spec(D_H), _full_spec((D_H, D)), _full_spec((1, D)),
    ],
    out_specs=_row_spec(D),
    out_shape=jax.ShapeDtypeStruct((N_PAD, D), jnp.float32),
)


def kernel(x, edge_index, W1_l, b1_l, W1_r, W2_l, b2_l, W2_r):
    src = edge_index[0]
    dst = edge_index[1]
    pad = E_PAD - src.shape[0]
    fill = jnp.arange(pad, dtype=jnp.int32)
    src_p = jnp.concatenate([src, fill % N_NODES])
    dst_p = jnp.concatenate([dst, N_NODES + fill % (N_PAD - N_NODES)])
    src3 = src_p.reshape(NW, GC, GROUP)
    dst3 = dst_p.reshape(NW, GC, GROUP)
    x_pad = jnp.zeros((N_PAD, D), jnp.float32).at[:N_NODES].set(x)

    cnts = _make_counts()(dst3)
    s1 = _make_seg_sum()(src3, dst3, x)
    h_pad, y_pad = _tc1(s1[0], s1[1], cnts[0], cnts[1], x_pad,
                        W1_l.T, b1_l.reshape(1, -1), W1_r.T, W2_l.T)
    s2 = _make_seg_sum()(src3, dst3, y_pad)
    out_pad = _tc2(s2[0], s2[1], cnts[0], cnts[1], h_pad,
                   W2_r.T, b2_l.reshape(1, -1))
    return out_pad[:N_NODES]


# speedup vs baseline: 3.3972x; 1.3404x over previous
import functools

import jax
import jax.numpy as jnp
from jax import lax
from jax.experimental import pallas as pl
from jax.experimental.pallas import tpu as pltpu
from jax.experimental.pallas import tpu_sc as plsc

N_NODES = 10000
D = 128
D_H = 256
NC, NS = 2, 16
NW = NC * NS
GROUP = 128
GC = 80
E_PAD = NW * GC * GROUP
N_PAD = 10240
RPT = N_PAD // NS
CW = 128


@functools.lru_cache(maxsize=None)
def _make_seg_sum():
    mesh = plsc.VectorSubcoreMesh(core_axis_name="c", subcore_axis_name="s")

    def body(pk_h, tab_h, out_h, pk_v, srcb, dstb, rows0, rows1,
             acc_sh, sem0, sem1):
        c = lax.axis_index("c")
        s = lax.axis_index("s")
        wid = c * NS + s

        zero16 = jnp.zeros((16,), jnp.float32)
        mask16 = jnp.full((16,), 0xFFFF, jnp.int32)

        @pl.loop(0, GROUP)
        def _(i):
            @pl.loop(0, D // 16)
            def _(j):
                rows0[i, pl.ds(j * 16, 16)] = zero16

        @pl.loop(0, RPT // GROUP)
        def _(j):
            base = s * RPT + j * GROUP
            pltpu.sync_copy(rows0, acc_sh.at[pl.ds(base, GROUP)])

        plsc.subcore_barrier()

        pltpu.sync_copy(pk_h.at[wid], pk_v)

        def unpack(g, slot):
            @pl.loop(0, GROUP // 16)
            def _(k):
                pk = pk_v[g, pl.ds(k * 16, 16)]
                srcb[slot, pl.ds(k * 16, 16)] = pk & mask16
                dstb[slot, pl.ds(k * 16, 16)] = lax.shift_right_logical(pk, 16)

        def gather(slot, buf, sem):
            pltpu.make_async_copy(tab_h.at[srcb.at[slot]], buf, sem).start()

        def gwait(slot, buf, sem):
            pltpu.make_async_copy(tab_h.at[srcb.at[slot]], buf, sem).wait()

        unpack(0, 0)
        unpack(1, 1)
        gather(0, rows0, sem0)
        gather(1, rows1, sem1)

        @pl.loop(0, GC // 2)
        def _(t):
            g0 = 2 * t
            gwait(0, rows0, sem0)
            pltpu.sync_copy(rows0, acc_sh.at[dstb.at[0]], add=True)
            unpack(jnp.minimum(g0 + 2, GC - 2), 0)
            gather(0, rows0, sem0)
            gwait(1, rows1, sem1)
            pltpu.sync_copy(rows1, acc_sh.at[dstb.at[1]], add=True)
            unpack(jnp.minimum(g0 + 3, GC - 1), 1)
            gather(1, rows1, sem1)

        gwait(0, rows0, sem0)
        gwait(1, rows1, sem1)

        plsc.subcore_barrier()

        @pl.loop(0, RPT // GROUP)
        def _(j):
            base = s * RPT + j * GROUP
            pltpu.sync_copy(acc_sh.at[pl.ds(base, GROUP)],
                            out_h.at[c].at[pl.ds(base, GROUP)])

    return pl.kernel(
        body,
        out_type=jax.ShapeDtypeStruct((NC, N_PAD, D), jnp.float32),
        mesh=mesh,
        scratch_types=[
            pltpu.VMEM((GC, GROUP), jnp.int32),
            pltpu.VMEM((2, GROUP), jnp.int32),
            pltpu.VMEM((2, GROUP), jnp.int32),
            pltpu.VMEM((GROUP, D), jnp.float32),
            pltpu.VMEM((GROUP, D), jnp.float32),
            pltpu.VMEM_SHARED((N_PAD, D), jnp.float32),
            pltpu.SemaphoreType.DMA,
            pltpu.SemaphoreType.DMA,
        ])


@functools.lru_cache(maxsize=None)
def _make_counts():
    mesh = plsc.VectorSubcoreMesh(core_axis_name="c", subcore_axis_name="s")

    def body(dst_h, cnt_h, dst_v, ones_v, cnt_sh):
        c = lax.axis_index("c")
        s = lax.axis_index("s")
        wid = c * NS + s

        @pl.loop(0, GROUP)
        def _(i):
            @pl.loop(0, CW // 16)
            def _(j):
                ones_v[i, pl.ds(j * 16, 16)] = jnp.zeros((16,), jnp.float32)

        @pl.loop(0, RPT // GROUP)
        def _(j):
            base = s * RPT + j * GROUP
            pltpu.sync_copy(ones_v, cnt_sh.at[pl.ds(base, GROUP)])

        @pl.loop(0, GROUP)
        def _(i):
            @pl.loop(0, CW // 16)
            def _(j):
                ones_v[i, pl.ds(j * 16, 16)] = jnp.ones((16,), jnp.float32)

        plsc.subcore_barrier()

        pltpu.sync_copy(dst_h.at[wid], dst_v)

        @pl.loop(0, GC)
        def _(g):
            pltpu.sync_copy(ones_v, cnt_sh.at[dst_v.at[g]], add=True)

        plsc.subcore_barrier()

        @pl.loop(0, RPT // GROUP)
        def _(j):
            base = s * RPT + j * GROUP
            pltpu.sync_copy(cnt_sh.at[pl.ds(base, GROUP)],
                            cnt_h.at[c].at[pl.ds(base, GROUP)])

    return pl.kernel(
        body,
        out_type=jax.ShapeDtypeStruct((NC, N_PAD, CW), jnp.float32),
        mesh=mesh,
        scratch_types=[
            pltpu.VMEM((GC, GROUP), jnp.int32),
            pltpu.VMEM((GROUP, CW), jnp.float32),
            pltpu.VMEM_SHARED((N_PAD, CW), jnp.float32),
        ])


BLK = 1024
NBLK = N_PAD // BLK

_DOT = functools.partial(jnp.dot, preferred_element_type=jnp.float32,
                         precision=jax.lax.Precision.HIGHEST)


def _tc1_body(s0, s1, c0, c1, x, w1l, b1, w1r, w2l, h_ref, y_ref):
    cnt = jnp.maximum(c0[...][:, 0:1] + c1[...][:, 0:1], 1.0)
    mean = (s0[...] + s1[...]) / cnt
    acc = _DOT(mean, w1l[...]) + _DOT(x[...], w1r[...]) + b1[...]
    h = jnp.maximum(acc, 0.0)
    h_ref[...] = h
    y_ref[...] = _DOT(h, w2l[...])


def _tc2_body(t0, t1, c0, c1, h, w2r, b2, o_ref):
    cnt = jnp.maximum(c0[...][:, 0:1] + c1[...][:, 0:1], 1.0)
    mean = (t0[...] + t1[...]) / cnt
    o_ref[...] = mean + b2[...] + _DOT(h[...], w2r[...])


def _row_spec(w):
    return pl.BlockSpec((BLK, w), lambda i: (i, 0))


def _full_spec(shape):
    return pl.BlockSpec(shape, lambda i: (0,) * len(shape))


_tc1 = pl.pallas_call(
    _tc1_body,
    grid=(NBLK,),
    in_specs=[
        _row_spec(D), _row_spec(D), _row_spec(CW), _row_spec(CW), _row_spec(D),
        _full_spec((D, D_H)), _full_spec((1, D_H)), _full_spec((D, D_H)),
        _full_spec((D_H, D)),
    ],
    out_specs=[_row_spec(D_H), _row_spec(D)],
    out_shape=[
        jax.ShapeDtypeStruct((N_PAD, D_H), jnp.float32),
        jax.ShapeDtypeStruct((N_PAD, D), jnp.float32),
    ],
)

_tc2 = pl.pallas_call(
    _tc2_body,
    grid=(NBLK,),
    in_specs=[
        _row_spec(D), _row_spec(D), _row_spec(CW), _row_spec(CW),
        _row_spec(D_H), _full_spec((D_H, D)), _full_spec((1, D)),
    ],
    out_specs=_row_spec(D),
    out_shape=jax.ShapeDtypeStruct((N_PAD, D), jnp.float32),
)


def kernel(x, edge_index, W1_l, b1_l, W1_r, W2_l, b2_l, W2_r):
    src = edge_index[0]
    dst = edge_index[1]
    pad = E_PAD - src.shape[0]
    fill = jnp.arange(pad, dtype=jnp.int32)
    src_p = jnp.concatenate([src, fill % N_NODES])
    dst_p = jnp.concatenate([dst, N_NODES + fill % (N_PAD - N_NODES)])
    pk3 = (src_p | (dst_p << 16)).reshape(NW, GC, GROUP)
    x_pad = jnp.zeros((N_PAD, D), jnp.float32).at[:N_NODES].set(x)

    cnts = _make_counts()(dst_p.reshape(NW, GC, GROUP))
    s1 = _make_seg_sum()(pk3, x)
    h_pad, y_pad = _tc1(s1[0], s1[1], cnts[0], cnts[1], x_pad,
                        W1_l.T, b1_l.reshape(1, -1), W1_r.T, W2_l.T)
    s2 = _make_seg_sum()(pk3, y_pad)
    out_pad = _tc2(s2[0], s2[1], cnts[0], cnts[1], h_pad,
                   W2_r.T, b2_l.reshape(1, -1))
    return out_pad[:N_NODES]
